# baseline (speedup 1.0000x reference)
import jax
import jax.numpy as jnp
from jax.experimental import pallas as pl

N = 100000
G = 512
STATE_DIM = 10
ROUNDS = 3


def _ident(x_ref, o_ref):
    o_ref[...] = x_ref[...]


def kernel(edge_lengths, node_from, node_to, node_graph_index, W1, b1, W2, b2):
    state = jnp.zeros((N, STATE_DIM), dtype=jnp.float32)
    for _ in range(ROUNDS):
        gathered = jnp.take(state, node_from, axis=0)
        inp = jnp.concatenate([gathered, edge_lengths], axis=1)
        message = jnp.tanh(inp @ W1.T + b1)
        state = state.at[node_to].add(message)
    graph_state = jax.ops.segment_sum(state, node_graph_index, num_segments=G)
    out = graph_state @ W2.T + b2
    return pl.pallas_call(
        _ident, out_shape=jax.ShapeDtypeStruct(out.shape, out.dtype)
    )(out)


# speedup vs baseline: 2.7432x; 2.7432x over previous
import functools

import jax
import jax.numpy as jnp
from jax import lax
from jax.experimental import pallas as pl
from jax.experimental.pallas import tpu as pltpu
from jax.experimental.pallas import tpu_sc as plsc

N = 100000
E = 6400000
G = 512
SD = 10
NC = 2
NS = 16
NW = NC * NS

NP = 100352
EPW = 401408
EP = EPW * NS
PADE = EP - E
CHUNK = 2048
KSUB = CHUNK // 128
NCHUNK = EPW // CHUNK
ROWS_PER_SUB = NP // NS
RPW = NP // NW
FLAT_PW = RPW * SD
FCH = 6272
GB = 520

_mesh = lambda: plsc.VectorSubcoreMesh(core_axis_name="c", subcore_axis_name="s")
_mesh1 = lambda: plsc.VectorSubcoreMesh(core_axis_name="c", subcore_axis_name="s",
                                        num_cores=1)
_cparams = lambda: pltpu.CompilerParams(needs_layout_passes=False,
                                        use_tc_tiling_on_sc=False)


def _make_round(has_gather):
    @functools.partial(
        pl.kernel,
        mesh=_mesh(),
        compiler_params=_cparams(),
        out_type=jax.ShapeDtypeStruct((NP, SD), jnp.float32),
        scratch_types=[
            pltpu.VMEM((192,), jnp.float32),
            pltpu.VMEM((KSUB, 128), jnp.int32),
            pltpu.VMEM((KSUB, 128), jnp.int32),
            pltpu.VMEM((CHUNK,), jnp.float32),
            pltpu.VMEM((CHUNK, 5), jnp.int32),
            pltpu.VMEM((CHUNK, SD), jnp.float32),
            pltpu.VMEM_SHARED((NP // 2 + 16, SD), jnp.float32),
            pltpu.SemaphoreType.DMA,
        ],
    )
    def round_kernel(state_hbm, state_pk, from_hbm, to0_hbm, to1_hbm,
                     len_hbm, blob_hbm, out_hbm, w_v, from_v, to_v, len_v,
                     rows_v, msgs_v, delta_sh, sem):
        c = lax.axis_index("c")
        s = lax.axis_index("s")
        w = s
        pltpu.sync_copy(blob_hbm, w_v)
        rsub = (NP // 2) // NS
        r0 = s * rsub

        pltpu.sync_copy(state_hbm.at[pl.ds(c * (NP // 2) + r0, rsub)],
                        delta_sh.at[pl.ds(r0, rsub)])

        plsc.subcore_barrier()

        iota16 = lax.iota(jnp.int32, 16)
        wvecs = [w_v[pl.ds(j * 16, 16)] for j in range(12)]
        cvec = wvecs[10]
        bvec = wvecs[11]
        col5 = jnp.minimum(iota16, 4)
        mask10 = iota16 < 10
        blk0 = w * (EPW // 128)
        NG = CHUNK // 16

        def it(q, carry):
            t = q // NG
            g2 = q % NG

            @pl.when(g2 == 0)
            def _():
                blk = blk0 + t * KSUB
                ebase = w * EPW + t * CHUNK
                if has_gather:
                    pltpu.sync_copy(from_hbm.at[pl.ds(blk, KSUB)], from_v)

                @pl.when(c == 0)
                def _():
                    pltpu.sync_copy(to0_hbm.at[pl.ds(blk, KSUB)], to_v)

                @pl.when(c != 0)
                def _():
                    pltpu.sync_copy(to1_hbm.at[pl.ds(blk, KSUB)], to_v)

                pltpu.sync_copy(len_hbm.at[pl.ds(ebase, CHUNK)], len_v)
                if has_gather:
                    cps = [pltpu.async_copy(state_pk.at[from_v.at[k]],
                                            rows_v.at[pl.ds(k * 128, 128)],
                                            sem)
                           for k in range(KSUB)]
                    for cp in cps:
                        cp.wait()

            len16 = len_v[pl.ds(g2 * 16, 16)]
            for l in range(16):
                e = g2 * 16 + l
                acc = len16[l] * cvec + bvec
                if has_gather:
                    w32v = plsc.load_gather(
                        rows_v, [jnp.full((16,), e, jnp.int32), col5])
                    glo = plsc.bitcast(lax.shift_left(w32v, 16), jnp.float32)
                    ghi = plsc.bitcast(
                        lax.bitwise_and(w32v, jnp.int32(-65536)), jnp.float32)
                    for k2 in range(5):
                        acc = acc + glo[k2] * wvecs[2 * k2]
                        acc = acc + ghi[k2] * wvecs[2 * k2 + 1]
                acc = jnp.minimum(jnp.maximum(acc, -50.0), 50.0)
                ex = jnp.exp(acc)
                m = (ex - 1.0) / (ex + 1.0)
                plsc.store_scatter(msgs_v,
                                   [jnp.full((16,), e, jnp.int32), iota16],
                                   m, mask=mask10)

            @pl.when(g2 == NG - 1)
            def _():
                for k in range(KSUB):
                    pltpu.sync_copy(msgs_v.at[pl.ds(k * 128, 128)],
                                    delta_sh.at[to_v.at[k]], add=True)

            return carry

        lax.fori_loop(0, NCHUNK * NG, it, 0)
        plsc.subcore_barrier()
        pltpu.sync_copy(delta_sh.at[pl.ds(r0, rsub)],
                        out_hbm.at[pl.ds(c * (NP // 2) + r0, rsub)])

    return round_kernel


_round_g = _make_round(True)


@functools.partial(
    pl.kernel,
    mesh=_mesh(),
    compiler_params=_cparams(),
    out_type=jax.ShapeDtypeStruct((NC, G, 2), jnp.float32),
    scratch_types=[
        pltpu.VMEM((32,), jnp.float32),
        pltpu.VMEM((RPW,), jnp.int32),
        pltpu.VMEM((FLAT_PW + 16,), jnp.float32),
        pltpu.VMEM((GB, 16), jnp.float32),
        pltpu.VMEM((4, 128), jnp.int32),
        pltpu.VMEM((32, 16), jnp.float32),
        pltpu.VMEM((32, 2), jnp.float32),
        pltpu.VMEM_SHARED((G, 16), jnp.float32),
    ],
)
def _final(s_hbm, gidx_hbm, w2_hbm, io_hbm, out_hbm,
           w2_v, gidx_v, s_v, gbuf, io_v, gv, obuf, gacc):
    c = lax.axis_index("c")
    s = lax.axis_index("s")
    w = c * NS + s
    pltpu.sync_copy(w2_hbm, w2_v)
    pltpu.sync_copy(io_hbm, io_v)
    pltpu.sync_copy(gidx_hbm.at[pl.ds(w * RPW, RPW)], gidx_v)
    pltpu.sync_copy(s_hbm.at[pl.ds(w * FLAT_PW, FLAT_PW)],
                    s_v.at[pl.ds(0, FLAT_PW)])
    iota16 = lax.iota(jnp.int32, 16)
    zv = jnp.zeros((16,), jnp.float32)

    def zb(i, cr):
        plsc.store_scatter(gbuf, [jnp.full((16,), i, jnp.int32), iota16], zv)
        return cr

    lax.fori_loop(0, GB, zb, 0)
    pltpu.sync_copy(gbuf.at[pl.ds(s * 32, 32)], gacc.at[pl.ds(s * 32, 32)])
    plsc.subcore_barrier()

    def acc_row(i, cr):
        g16 = gidx_v[pl.ds(i * 16, 16)]
        for l in range(16):
            g = g16[l]
            row = s_v[pl.ds(i * 16 * SD + l * SD, 16)]
            plsc.addupdate_scatter(
                gbuf, [jnp.full((16,), g, jnp.int32), iota16], row)
        return cr

    lax.fori_loop(0, RPW // 16, acc_row, 0)
    for k in range(4):
        pltpu.sync_copy(gbuf.at[pl.ds(k * 128, 128)], gacc.at[io_v.at[k]],
                        add=True)
    plsc.subcore_barrier()
    pltpu.sync_copy(gacc.at[pl.ds(s * 32, 32)], gv)
    w2l = [w2_v[pl.ds(0, 16)], w2_v[pl.ds(16, 16)]]
    w2sc = lambda idx: w2l[idx // 16][idx % 16]
    for q in range(2):
        rid = q * 16 + iota16
        gsv = [plsc.load_gather(gv, [rid, jnp.full((16,), j, jnp.int32)])
               for j in range(SD)]
        o0 = gsv[0] * w2sc(0)
        o1 = gsv[0] * w2sc(SD)
        for j in range(1, SD):
            o0 = o0 + gsv[j] * w2sc(j)
            o1 = o1 + gsv[j] * w2sc(SD + j)
        plsc.store_scatter(obuf, [rid, jnp.full((16,), 0, jnp.int32)], o0)
        plsc.store_scatter(obuf, [rid, jnp.full((16,), 1, jnp.int32)], o1)
    pltpu.sync_copy(obuf, out_hbm.at[c, pl.ds(s * 32, 32)])


def kernel(edge_lengths, node_from, node_to, node_graph_index, W1, b1, W2, b2):
    f32 = jnp.float32
    nf = node_from.astype(jnp.int32)
    nt = node_to.astype(jnp.int32)
    gi = node_graph_index.astype(jnp.int32)
    el = edge_lengths.reshape(E).astype(f32)

    pad_from = lax.iota(jnp.int32, PADE) % N
    pad_to = N + lax.iota(jnp.int32, PADE) % (NP - N)
    nf2 = jnp.concatenate([nf, pad_from]).reshape(EP // 128, 128)
    ntp = jnp.concatenate([nt, pad_to])
    H = NP // 2
    trash = H + (ntp % 16)
    nt0 = jnp.where(ntp < H, ntp, trash).reshape(EP // 128, 128)
    nt1 = jnp.where(ntp >= H, ntp - H, trash).reshape(EP // 128, 128)
    elp = jnp.concatenate([el, jnp.zeros((PADE,), f32)])
    gip = jnp.concatenate([gi, jnp.full((NP - N,), G, jnp.int32)])

    blob = jnp.zeros((12, 16), f32)
    blob = blob.at[:SD + 1, :SD].set(2.0 * W1.astype(f32).T)
    blob = blob.at[11, :SD].set(2.0 * b1.astype(f32))
    blob = blob.reshape(192)
    w2b = jnp.zeros((32,), f32)
    w2b = w2b.at[:SD].set(W2[0].astype(f32)).at[SD:2 * SD].set(W2[1].astype(f32))
    io512 = jnp.arange(G, dtype=jnp.int32).reshape(4, 128)

    def body(_, st):
        st_pk = lax.bitcast_convert_type(
            st.reshape(NP, 5, 2).astype(jnp.bfloat16), jnp.int32)
        return _round_g(st, st_pk, nf2, nt0, nt1, elp, blob)

    st = lax.fori_loop(0, 3, body, jnp.zeros((NP, SD), f32))
    pout = _final(st.reshape(NP * SD), gip, w2b, io512)
    return pout[0] + pout[1] + b2[None, :].astype(f32)
